# baseline (speedup 1.0000x reference)
import functools

import jax
import jax.numpy as jnp
from jax import lax
from jax.experimental import pallas as pl
from jax.experimental.pallas import tpu as pltpu
from jax.experimental.pallas import tpu_sc as plsc

NUM_NODES = 1000000
MEMORY_DIM = 64
BATCH = 16384

_LOOK = 3
_RING = 4
_WCHUNK = 64


def _make_gather():
    info = plsc.get_sparse_core_info()
    nc, ns = info.num_cores, info.num_subcores
    nw = nc * ns
    b_per_w = BATCH // nw
    mesh = plsc.VectorSubcoreMesh(core_axis_name="c", subcore_axis_name="s")

    @functools.partial(
        pl.kernel,
        mesh=mesh,
        out_type=jax.ShapeDtypeStruct((BATCH, MEMORY_DIM), jnp.float32),
        scratch_types=[
            pltpu.VMEM((b_per_w,), jnp.int32),
            pltpu.VMEM((_RING, MEMORY_DIM, 128), jnp.float32),
            pltpu.VMEM((2 * _WCHUNK, MEMORY_DIM), jnp.float32),
            pltpu.SemaphoreType.DMA((_RING,)),
        ],
        compiler_params=pltpu.CompilerParams(needs_layout_passes=False),
    )
    def gather_kernel(table_hbm, ids_hbm, out_hbm, idx_v, bufs, comp, sems):
        wid = lax.axis_index("s") * nc + lax.axis_index("c")
        base = wid * b_per_w
        pltpu.sync_copy(ids_hbm.at[pl.ds(base, b_per_w)], idx_v)

        iota = lax.iota(jnp.int32, 16)
        zeros16 = jnp.zeros((16,), jnp.int32)
        kvecs = [iota + 16 * g for g in range(MEMORY_DIM // 16)]

        def id_scalar(f):
            w16 = idx_v[pl.ds(f & ~15, 16)]
            return jnp.sum(jnp.where(iota == (f & 15), w16, 0))

        def fire(f):
            j = id_scalar(f)
            off = pl.multiple_of((j >> 7) * 128, 128)
            pltpu.async_copy(
                table_hbm.at[:, :, pl.ds(off, 128)],
                bufs.at[pl.ds(f % _RING, 1)],
                sems.at[f % _RING],
            )

        def wait_slot(i):
            pltpu.make_async_copy(
                table_hbm.at[:, :, pl.ds(0, 128)],
                bufs.at[pl.ds(i % _RING, 1)],
                sems.at[i % _RING],
            ).wait()

        for f in range(_LOOK):
            fire(f)

        def body(i, carry):
            @pl.when(i + _LOOK < b_per_w)
            def _():
                fire(i + _LOOK)

            wait_slot(i)
            j = id_scalar(i)
            c_splat = zeros16 + (j & 127)
            slot_splat = zeros16 + (i % _RING)
            row_splat = zeros16 + (i % (2 * _WCHUNK))
            for kv in kvecs:
                val = plsc.load_gather(bufs, [slot_splat, kv, c_splat])
                plsc.store_scatter(comp, [row_splat, kv], val)

            @pl.when((i % _WCHUNK) == (_WCHUNK - 1))
            def _():
                start = pl.multiple_of(i & _WCHUNK, _WCHUNK)
                dst = pl.multiple_of(base + (i - (_WCHUNK - 1)), _WCHUNK)
                pltpu.sync_copy(
                    comp.at[pl.ds(start, _WCHUNK)],
                    out_hbm.at[pl.ds(dst, _WCHUNK)],
                )

            return carry

        lax.fori_loop(0, b_per_w, body, 0)

    return gather_kernel, nw


def kernel(node_ids, memory):
    gather_kernel, nw = _make_gather()
    table3 = memory.T.reshape(1, MEMORY_DIM, NUM_NODES)
    return gather_kernel(table3, node_ids)


# speedup vs baseline: 1.0278x; 1.0278x over previous
import functools

import jax
import jax.numpy as jnp
from jax import lax
from jax.experimental import pallas as pl
from jax.experimental.pallas import tpu as pltpu
from jax.experimental.pallas import tpu_sc as plsc

NUM_NODES = 1000000
MEMORY_DIM = 64
BATCH = 16384

_LOOK = 3
_RING = 4
_WCHUNK = 128


def _make_gather():
    info = plsc.get_sparse_core_info()
    nc, ns = info.num_cores, info.num_subcores
    nw = nc * ns
    b_per_w = BATCH // nw
    mesh = plsc.VectorSubcoreMesh(core_axis_name="c", subcore_axis_name="s")

    @functools.partial(
        pl.kernel,
        mesh=mesh,
        out_type=jax.ShapeDtypeStruct((MEMORY_DIM, BATCH), jnp.float32),
        scratch_types=[
            pltpu.VMEM((b_per_w,), jnp.int32),
            pltpu.VMEM((_RING, MEMORY_DIM, 128), jnp.float32),
            pltpu.VMEM((MEMORY_DIM, 2 * _WCHUNK), jnp.float32),
            pltpu.SemaphoreType.DMA((_RING,)),
        ],
        compiler_params=pltpu.CompilerParams(needs_layout_passes=False),
    )
    def gather_kernel(table_hbm, ids_hbm, out_hbm, idx_v, bufs, comp, sems):
        wid = lax.axis_index("s") * nc + lax.axis_index("c")
        base = wid * b_per_w
        pltpu.sync_copy(ids_hbm.at[pl.ds(base, b_per_w)], idx_v)

        iota = lax.iota(jnp.int32, 16)
        zeros16 = jnp.zeros((16,), jnp.int32)
        kvecs = [iota + 16 * g for g in range(MEMORY_DIM // 16)]

        def id_scalar(f):
            w16 = idx_v[pl.ds(f & ~15, 16)]
            return jnp.sum(jnp.where(iota == (f & 15), w16, 0))

        def fire(f):
            j = id_scalar(f)
            off = pl.multiple_of((j >> 7) * 128, 128)
            pltpu.async_copy(
                table_hbm.at[:, :, pl.ds(off, 128)],
                bufs.at[pl.ds(f % _RING, 1)],
                sems.at[f % _RING],
            )

        def wait_slot(i):
            pltpu.make_async_copy(
                table_hbm.at[:, :, pl.ds(0, 128)],
                bufs.at[pl.ds(i % _RING, 1)],
                sems.at[i % _RING],
            ).wait()

        for f in range(_LOOK):
            fire(f)

        def body(i, carry):
            @pl.when(i + _LOOK < b_per_w)
            def _():
                fire(i + _LOOK)

            wait_slot(i)
            j = id_scalar(i)
            c_splat = zeros16 + (j & 127)
            slot_splat = zeros16 + (i % _RING)
            col_splat = zeros16 + (i % (2 * _WCHUNK))
            for kv in kvecs:
                val = plsc.load_gather(bufs, [slot_splat, kv, c_splat])
                plsc.store_scatter(comp, [kv, col_splat], val)

            @pl.when((i % _WCHUNK) == (_WCHUNK - 1))
            def _():
                start = pl.multiple_of(i & _WCHUNK, _WCHUNK)
                dst = pl.multiple_of(base + (i - (_WCHUNK - 1)), _WCHUNK)
                pltpu.sync_copy(
                    comp.at[:, pl.ds(start, _WCHUNK)],
                    out_hbm.at[:, pl.ds(dst, _WCHUNK)],
                )

            return carry

        lax.fori_loop(0, b_per_w, body, 0)

    return gather_kernel, nw


def kernel(node_ids, memory):
    gather_kernel, nw = _make_gather()
    table3 = memory.T.reshape(1, MEMORY_DIM, NUM_NODES)
    return gather_kernel(table3, node_ids).T


# speedup vs baseline: 1.1221x; 1.0917x over previous
import functools

import jax
import jax.numpy as jnp
from jax import lax
from jax.experimental import pallas as pl
from jax.experimental.pallas import tpu as pltpu
from jax.experimental.pallas import tpu_sc as plsc

NUM_NODES = 1000000
MEMORY_DIM = 64
BATCH = 16384

_LOOK = 5
_RING = 6
_WCHUNK = 128


def _make_gather():
    info = plsc.get_sparse_core_info()
    nc, ns = info.num_cores, info.num_subcores
    nw = nc * ns
    b_per_w = BATCH // nw
    mesh = plsc.VectorSubcoreMesh(core_axis_name="c", subcore_axis_name="s")

    @functools.partial(
        pl.kernel,
        mesh=mesh,
        out_type=jax.ShapeDtypeStruct((MEMORY_DIM, BATCH), jnp.float32),
        scratch_types=[
            pltpu.VMEM((b_per_w,), jnp.int32),
            pltpu.VMEM((_RING, MEMORY_DIM, 128), jnp.float32),
            pltpu.VMEM((MEMORY_DIM, 2 * _WCHUNK), jnp.float32),
            pltpu.SemaphoreType.DMA((_RING,)),
        ],
        compiler_params=pltpu.CompilerParams(needs_layout_passes=False),
    )
    def gather_kernel(table_hbm, ids_hbm, out_hbm, idx_v, bufs, comp, sems):
        wid = lax.axis_index("s") * nc + lax.axis_index("c")
        base = wid * b_per_w
        pltpu.sync_copy(ids_hbm.at[pl.ds(base, b_per_w)], idx_v)

        iota = lax.iota(jnp.int32, 16)
        zeros16 = jnp.zeros((16,), jnp.int32)
        kvecs = [iota + 16 * g for g in range(MEMORY_DIM // 16)]

        def id_scalar(f):
            w16 = idx_v[pl.ds(f & ~15, 16)]
            return jnp.sum(jnp.where(iota == (f & 15), w16, 0))

        def fire(f):
            j = id_scalar(f)
            off = pl.multiple_of((j >> 7) * 128, 128)
            pltpu.async_copy(
                table_hbm.at[:, :, pl.ds(off, 128)],
                bufs.at[pl.ds(f % _RING, 1)],
                sems.at[f % _RING],
            )

        def wait_slot(i):
            pltpu.make_async_copy(
                table_hbm.at[:, :, pl.ds(0, 128)],
                bufs.at[pl.ds(i % _RING, 1)],
                sems.at[i % _RING],
            ).wait()

        for f in range(_LOOK):
            fire(f)

        def body(i, carry):
            @pl.when(i + _LOOK < b_per_w)
            def _():
                fire(i + _LOOK)

            wait_slot(i)
            j = id_scalar(i)
            c_splat = zeros16 + (j & 127)
            slot_splat = zeros16 + (i % _RING)
            col_splat = zeros16 + (i % (2 * _WCHUNK))
            for kv in kvecs:
                val = plsc.load_gather(bufs, [slot_splat, kv, c_splat])
                plsc.store_scatter(comp, [kv, col_splat], val)

            @pl.when((i % _WCHUNK) == (_WCHUNK - 1))
            def _():
                start = pl.multiple_of(i & _WCHUNK, _WCHUNK)
                dst = pl.multiple_of(base + (i - (_WCHUNK - 1)), _WCHUNK)
                pltpu.sync_copy(
                    comp.at[:, pl.ds(start, _WCHUNK)],
                    out_hbm.at[:, pl.ds(dst, _WCHUNK)],
                )

            return carry

        lax.fori_loop(0, b_per_w, body, 0)

    return gather_kernel, nw


def kernel(node_ids, memory):
    gather_kernel, nw = _make_gather()
    table3 = memory.T.reshape(1, MEMORY_DIM, NUM_NODES)
    return gather_kernel(table3, node_ids).T


# speedup vs baseline: 1.2056x; 1.0745x over previous
import functools

import jax
import jax.numpy as jnp
from jax import lax
from jax.experimental import pallas as pl
from jax.experimental.pallas import tpu as pltpu
from jax.experimental.pallas import tpu_sc as plsc

NUM_NODES = 1000000
MEMORY_DIM = 64
BATCH = 16384

_LOOK = 9
_RING = 10
_WCHUNK = 128


def _make_gather():
    info = plsc.get_sparse_core_info()
    nc, ns = info.num_cores, info.num_subcores
    nw = nc * ns
    b_per_w = BATCH // nw
    mesh = plsc.VectorSubcoreMesh(core_axis_name="c", subcore_axis_name="s")

    @functools.partial(
        pl.kernel,
        mesh=mesh,
        out_type=jax.ShapeDtypeStruct((MEMORY_DIM, BATCH), jnp.float32),
        scratch_types=[
            pltpu.VMEM((b_per_w,), jnp.int32),
            pltpu.VMEM((_RING, MEMORY_DIM, 128), jnp.float32),
            pltpu.VMEM((MEMORY_DIM, 2 * _WCHUNK), jnp.float32),
            pltpu.SemaphoreType.DMA((_RING,)),
        ],
        compiler_params=pltpu.CompilerParams(needs_layout_passes=False),
    )
    def gather_kernel(table_hbm, ids_hbm, out_hbm, idx_v, bufs, comp, sems):
        wid = lax.axis_index("s") * nc + lax.axis_index("c")
        base = wid * b_per_w
        pltpu.sync_copy(ids_hbm.at[pl.ds(base, b_per_w)], idx_v)

        iota = lax.iota(jnp.int32, 16)
        zeros16 = jnp.zeros((16,), jnp.int32)
        kvecs = [iota + 16 * g for g in range(MEMORY_DIM // 16)]

        def id_scalar(f):
            w16 = idx_v[pl.ds(f & ~15, 16)]
            return jnp.sum(jnp.where(iota == (f & 15), w16, 0))

        def fire(f):
            j = id_scalar(f)
            off = pl.multiple_of((j >> 7) * 128, 128)
            pltpu.async_copy(
                table_hbm.at[:, :, pl.ds(off, 128)],
                bufs.at[pl.ds(f % _RING, 1)],
                sems.at[f % _RING],
            )

        def wait_slot(i):
            pltpu.make_async_copy(
                table_hbm.at[:, :, pl.ds(0, 128)],
                bufs.at[pl.ds(i % _RING, 1)],
                sems.at[i % _RING],
            ).wait()

        for f in range(_LOOK):
            fire(f)

        def body(i, carry):
            @pl.when(i + _LOOK < b_per_w)
            def _():
                fire(i + _LOOK)

            wait_slot(i)
            j = id_scalar(i)
            c_splat = zeros16 + (j & 127)
            slot_splat = zeros16 + (i % _RING)
            col_splat = zeros16 + (i % (2 * _WCHUNK))
            for kv in kvecs:
                val = plsc.load_gather(bufs, [slot_splat, kv, c_splat])
                plsc.store_scatter(comp, [kv, col_splat], val)

            @pl.when((i % _WCHUNK) == (_WCHUNK - 1))
            def _():
                start = pl.multiple_of(i & _WCHUNK, _WCHUNK)
                dst = pl.multiple_of(base + (i - (_WCHUNK - 1)), _WCHUNK)
                pltpu.sync_copy(
                    comp.at[:, pl.ds(start, _WCHUNK)],
                    out_hbm.at[:, pl.ds(dst, _WCHUNK)],
                )

            return carry

        lax.fori_loop(0, b_per_w, body, 0)

    return gather_kernel, nw


def kernel(node_ids, memory):
    gather_kernel, nw = _make_gather()
    table3 = memory.T.reshape(1, MEMORY_DIM, NUM_NODES)
    return gather_kernel(table3, node_ids).T
